# Initial kernel scaffold; baseline (speedup 1.0000x reference)
import jax
import jax.numpy as jnp
from jax.experimental import pallas as pl


def kernel(src, time, weekday, duration, emb_loc_W, minute_W, hour_W, weekday_W, duration_W):
    raise NotImplementedError("write your pallas kernel here")


# speedup vs baseline: 8.3412x; 8.3412x over previous
import functools

import jax
import jax.numpy as jnp
from jax import lax
from jax.experimental import pallas as pl
from jax.experimental.pallas import tpu as pltpu
from jax.experimental.pallas import tpu_sc as plsc

D = 64
NC, NS = 2, 16
NW = NC * NS
C = 1024
G = 128


def _comb_body(minute_ref, hour_ref, weekday_ref, duration_ref, out_ref):
    hm = (jnp.broadcast_to(hour_ref[:][:, None, :], (24, 4, D))
          + jnp.broadcast_to(minute_ref[:][None, :, :], (24, 4, D))).reshape(96, D)
    row = lax.broadcasted_iota(jnp.int32, (7, D), 0) == pl.program_id(0)
    w = jnp.sum(jnp.where(row, weekday_ref[:], 0.0), axis=0)
    out_ref[0] = (hm[:, None, :] + duration_ref[:][None, :, :]
                  + w[None, None, :])


def _build_comb(minute_W, hour_W, weekday_W, duration_W):
    out = pl.pallas_call(
        _comb_body,
        grid=(7,),
        in_specs=[
            pl.BlockSpec((4, D), lambda w: (0, 0)),
            pl.BlockSpec((24, D), lambda w: (0, 0)),
            pl.BlockSpec((7, D), lambda w: (0, 0)),
            pl.BlockSpec((96, D), lambda w: (0, 0)),
        ],
        out_specs=pl.BlockSpec((1, 96, 96, D), lambda w: (w, 0, 0, 0)),
        out_shape=jax.ShapeDtypeStruct((7, 96, 96, D), jnp.float32),
    )(minute_W, hour_W, weekday_W, duration_W)
    return out.reshape(7 * 96 * 96, D)


def _make_sc_lookup(n_tokens):
    rpw = n_tokens // NW
    nchunk = rpw // C
    mesh = plsc.VectorSubcoreMesh(core_axis_name="c", subcore_axis_name="s")

    @functools.partial(
        pl.kernel,
        mesh=mesh,
        out_type=jax.ShapeDtypeStruct((n_tokens, D), jnp.float32),
        scratch_types=[
            pltpu.VMEM((C,), jnp.int32),
            pltpu.VMEM((C,), jnp.int32),
            pltpu.VMEM((C,), jnp.int32),
            pltpu.VMEM((C,), jnp.int32),
            pltpu.VMEM((C,), jnp.int32),
            pltpu.VMEM((C, D), jnp.float32),
            pltpu.SemaphoreType.DMA,
        ],
        compiler_params=pltpu.CompilerParams(use_tc_tiling_on_sc=False),
    )
    def sc_lookup(src_h, time_h, wk_h, dur_h, comb_h, loc_h, out_h,
                  sbuf, tbuf, wbuf, dbuf, cbuf, rows, sem):
        cid = lax.axis_index("c")
        sid = lax.axis_index("s")
        wid = sid * NC + cid

        @pl.loop(0, nchunk)
        def _chunk(k):
            base = wid * rpw + k * C
            pltpu.sync_copy(src_h.at[pl.ds(base, C)], sbuf)
            pltpu.sync_copy(time_h.at[pl.ds(base, C)], tbuf)
            pltpu.sync_copy(wk_h.at[pl.ds(base, C)], wbuf)
            pltpu.sync_copy(dur_h.at[pl.ds(base, C)], dbuf)

            @pl.loop(0, C // 16)
            def _cidx(i):
                sl = pl.ds(i * 16, 16)
                cbuf[sl] = (wbuf[sl] * 96 + tbuf[sl]) * 96 + dbuf[sl]

            descs = []
            for j in range(C // G):
                descs.append(pltpu.async_copy(
                    loc_h.at[sbuf.at[pl.ds(j * G, G)]],
                    rows.at[pl.ds(j * G, G)], sem))
            for d_ in descs:
                d_.wait()
            descs = []
            for j in range(C // G):
                descs.append(pltpu.async_copy(
                    comb_h.at[cbuf.at[pl.ds(j * G, G)]],
                    rows.at[pl.ds(j * G, G)], sem, add=True))
            for d_ in descs:
                d_.wait()
            pltpu.sync_copy(rows, out_h.at[pl.ds(base, C)])

    return sc_lookup


def kernel(src, time, weekday, duration, emb_loc_W, minute_W, hour_W,
           weekday_W, duration_W):
    B, L = src.shape
    n = B * L
    comb = _build_comb(minute_W, hour_W, weekday_W, duration_W)
    out = _make_sc_lookup(n)(
        src.reshape(n).astype(jnp.int32),
        time.reshape(n).astype(jnp.int32),
        weekday.reshape(n).astype(jnp.int32),
        duration.reshape(n).astype(jnp.int32),
        comb,
        emb_loc_W,
    )
    return out.reshape(B, L, D)
